# baseline (speedup 1.0000x reference)
import jax
import jax.numpy as jnp
from jax import lax
from jax.experimental import pallas as pl
from jax.experimental.pallas import tpu as pltpu

_NBUF = 5


def _body(ts_ref, gam_ref, x_hbm, n_hbm, o_hbm, xb, nb, ob, xsem, nsem, osem):
    nchunks = x_hbm.shape[0]

    def start_in(i, slot):
        pltpu.make_async_copy(x_hbm.at[i], xb.at[slot], xsem.at[slot]).start()
        pltpu.make_async_copy(n_hbm.at[i], nb.at[slot], nsem.at[slot]).start()

    for i in range(_NBUF):
        start_in(i, i)

    def step(i, _):
        slot = lax.rem(i, _NBUF)
        pltpu.make_async_copy(x_hbm.at[i], xb.at[slot], xsem.at[slot]).wait()
        pltpu.make_async_copy(n_hbm.at[i], nb.at[slot], nsem.at[slot]).wait()

        @pl.when(i >= _NBUF)
        def _():
            pltpu.make_async_copy(
                ob.at[slot], o_hbm.at[i - _NBUF], osem.at[slot]
            ).wait()

        g = gam_ref[ts_ref[i]]
        ob[slot] = jnp.sqrt(g) * xb[slot] + jnp.sqrt(1.0 - g) * nb[slot]
        pltpu.make_async_copy(ob.at[slot], o_hbm.at[i], osem.at[slot]).start()

        @pl.when(i + _NBUF < nchunks)
        def _():
            start_in(i + _NBUF, slot)

        return 0

    lax.fori_loop(0, nchunks, step, 0)

    def drain(i, _):
        slot = lax.rem(i, _NBUF)
        pltpu.make_async_copy(ob.at[slot], o_hbm.at[i], osem.at[slot]).wait()
        return 0

    lax.fori_loop(nchunks - _NBUF, nchunks, drain, 0)


def kernel(x_start, timesteps, noise, gammas):
    B, C, H, W = x_start.shape
    ts = timesteps.reshape(B).astype(jnp.int32)

    return pl.pallas_call(
        _body,
        grid=(),
        in_specs=[
            pl.BlockSpec(memory_space=pltpu.SMEM),
            pl.BlockSpec(memory_space=pltpu.SMEM),
            pl.BlockSpec(memory_space=pltpu.HBM),
            pl.BlockSpec(memory_space=pltpu.HBM),
        ],
        out_specs=pl.BlockSpec(memory_space=pltpu.HBM),
        scratch_shapes=[
            pltpu.VMEM((_NBUF, C, H, W), jnp.float32),
            pltpu.VMEM((_NBUF, C, H, W), jnp.float32),
            pltpu.VMEM((_NBUF, C, H, W), jnp.float32),
            pltpu.SemaphoreType.DMA((_NBUF,)),
            pltpu.SemaphoreType.DMA((_NBUF,)),
            pltpu.SemaphoreType.DMA((_NBUF,)),
        ],
        out_shape=jax.ShapeDtypeStruct((B, C, H, W), jnp.float32),
    )(ts, gammas.astype(jnp.float32), x_start, noise)


# speedup vs baseline: 1.0040x; 1.0040x over previous
import jax
import jax.numpy as jnp
from jax import lax
from jax.experimental import pallas as pl
from jax.experimental.pallas import tpu as pltpu

_NBUF = 8


def _body(ts_ref, gam_ref, x_hbm, n_hbm, o_hbm, xb, nb, ob, xsem, nsem, osem):
    B, C = x_hbm.shape[0], x_hbm.shape[1]
    nchunks = B * C

    def start_in(i, slot):
        b, c = lax.div(i, C), lax.rem(i, C)
        pltpu.make_async_copy(x_hbm.at[b, c], xb.at[slot], xsem.at[slot]).start()
        pltpu.make_async_copy(n_hbm.at[b, c], nb.at[slot], nsem.at[slot]).start()

    for i in range(_NBUF):
        start_in(i, i)

    def step(i, _):
        slot = lax.rem(i, _NBUF)
        b, c = lax.div(i, C), lax.rem(i, C)
        pltpu.make_async_copy(x_hbm.at[b, c], xb.at[slot], xsem.at[slot]).wait()
        pltpu.make_async_copy(n_hbm.at[b, c], nb.at[slot], nsem.at[slot]).wait()

        @pl.when(i >= _NBUF)
        def _():
            bp, cp = lax.div(i - _NBUF, C), lax.rem(i - _NBUF, C)
            pltpu.make_async_copy(
                ob.at[slot], o_hbm.at[bp, cp], osem.at[slot]
            ).wait()

        g = gam_ref[ts_ref[b]]
        ob[slot] = jnp.sqrt(g) * xb[slot] + jnp.sqrt(1.0 - g) * nb[slot]
        pltpu.make_async_copy(ob.at[slot], o_hbm.at[b, c], osem.at[slot]).start()

        @pl.when(i + _NBUF < nchunks)
        def _():
            start_in(i + _NBUF, slot)

        return 0

    lax.fori_loop(0, nchunks, step, 0)

    def drain(i, _):
        slot = lax.rem(i, _NBUF)
        b, c = lax.div(i, C), lax.rem(i, C)
        pltpu.make_async_copy(ob.at[slot], o_hbm.at[b, c], osem.at[slot]).wait()
        return 0

    lax.fori_loop(nchunks - _NBUF, nchunks, drain, 0)


def kernel(x_start, timesteps, noise, gammas):
    B, C, H, W = x_start.shape
    ts = timesteps.reshape(B).astype(jnp.int32)

    return pl.pallas_call(
        _body,
        grid=(),
        in_specs=[
            pl.BlockSpec(memory_space=pltpu.SMEM),
            pl.BlockSpec(memory_space=pltpu.SMEM),
            pl.BlockSpec(memory_space=pltpu.HBM),
            pl.BlockSpec(memory_space=pltpu.HBM),
        ],
        out_specs=pl.BlockSpec(memory_space=pltpu.HBM),
        scratch_shapes=[
            pltpu.VMEM((_NBUF, H, W), jnp.float32),
            pltpu.VMEM((_NBUF, H, W), jnp.float32),
            pltpu.VMEM((_NBUF, H, W), jnp.float32),
            pltpu.SemaphoreType.DMA((_NBUF,)),
            pltpu.SemaphoreType.DMA((_NBUF,)),
            pltpu.SemaphoreType.DMA((_NBUF,)),
        ],
        out_shape=jax.ShapeDtypeStruct((B, C, H, W), jnp.float32),
    )(ts, gammas.astype(jnp.float32), x_start, noise)
